# baseline (speedup 1.0000x reference)
import functools

import jax
import jax.numpy as jnp
from jax import lax
from jax.experimental import pallas as pl
from jax.experimental.pallas import tpu as pltpu
from jax.experimental.pallas import tpu_sc as plsc

N = 2048
B = 4
C = 1024
K = 512
NB = N * B
BK = B * K
NW = 32
RPW = BK // NW
CPB = NW // B
ZPW = NB // NW

_f32 = jnp.float32
_i32 = jnp.int32


def _sc_mesh():
    return plsc.VectorSubcoreMesh(core_axis_name="c", subcore_axis_name="s")


_ACH = 32


def _asm_body(x2_hbm, out_hbm, zbuf, buf0, buf1, semz, semr, semw):
    wid = lax.axis_index("c") * 16 + lax.axis_index("s")
    base = wid * ZPW

    def zrow(r, carry):
        for l in range(C // 16):
            zbuf[r, pl.ds(l * 16, 16)] = jnp.zeros((16,), _f32)
        return carry

    lax.fori_loop(0, _ACH, zrow, 0)
    zcopies = [
        pltpu.async_copy(
            zbuf, out_hbm.at[pl.ds(NB + base + c * _ACH, _ACH)], semz)
        for c in range(ZPW // _ACH)
    ]
    bufs = [buf0, buf1]
    wrs = [None] * (ZPW // _ACH)
    for c in range(ZPW // _ACH):
        if c >= 2:
            wrs[c - 2].wait()
        pltpu.async_copy(
            x2_hbm.at[pl.ds(base + c * _ACH, _ACH)], bufs[c % 2], semr).wait()
        wrs[c] = pltpu.async_copy(
            bufs[c % 2], out_hbm.at[pl.ds(base + c * _ACH, _ACH)], semw)
    wrs[-2].wait()
    wrs[-1].wait()
    for cp in zcopies:
        cp.wait()


@functools.cache
def _asm_call():
    return pl.kernel(
        _asm_body,
        out_type=jax.ShapeDtypeStruct((2 * NB, C), _f32),
        mesh=_sc_mesh(),
        compiler_params=pltpu.CompilerParams(needs_layout_passes=False),
        scratch_types=[
            pltpu.VMEM((_ACH, C), _f32),
            pltpu.VMEM((_ACH, C), _f32),
            pltpu.VMEM((_ACH, C), _f32),
            pltpu.SemaphoreType.DMA,
            pltpu.SemaphoreType.DMA,
            pltpu.SemaphoreType.DMA,
        ],
    )


def _rank_body(s_row_ref, s_col_ref, m_row_ref, m_col_ref, rank_ref):
    neg = _f32(-jnp.inf)
    s = jnp.where(m_row_ref[0], neg, s_row_ref[0])
    sc = jnp.where(m_col_ref[0], neg, s_col_ref[0])
    jj = lax.broadcasted_iota(_i32, (1, N), 1)
    ones = jnp.ones((N, 128), _f32)
    CH = 256
    for ci in range(N // CH):
        sic = sc[ci * CH:(ci + 1) * CH, :]
        ii = lax.broadcasted_iota(_i32, (CH, 1), 0) + ci * CH
        beats = (s > sic) | ((s == sic) & (jj < ii))
        bb = beats.astype(_f32)
        cnt = lax.dot_general(bb, ones, (((1,), (0,)), ((), ())),
                              preferred_element_type=_f32)
        rank_ref[0, ci * CH:(ci + 1) * CH, :] = cnt[:, 0:1].astype(_i32)


_rank_call = pl.pallas_call(
    _rank_body,
    grid=(B,),
    in_specs=[
        pl.BlockSpec((1, 1, N), lambda i: (i, 0, 0)),
        pl.BlockSpec((1, N, 1), lambda i: (i, 0, 0)),
        pl.BlockSpec((1, 1, N), lambda i: (i, 0, 0)),
        pl.BlockSpec((1, N, 1), lambda i: (i, 0, 0)),
    ],
    out_specs=pl.BlockSpec((1, N, 1), lambda i: (i, 0, 0)),
    out_shape=jax.ShapeDtypeStruct((B, N, 1), _i32),
)


def _gather_body(rank_hbm, x2_hbm, xg_hbm, self_hbm, rank_v, idx_v, rows_v, sem):
    wid = lax.axis_index("c") * 16 + lax.axis_index("s")
    b = wid // CPB
    lo = (wid % CPB) * RPW
    pltpu.sync_copy(rank_hbm.at[b], rank_v)
    lane = lax.iota(_i32, 16)

    def step(j, carry):
        r = rank_v[pl.ds(j * 16, 16)]
        tok = lane + j * 16
        m = (r >= lo) & (r < lo + RPW)
        plsc.store_scatter(idx_v, [r - lo], tok * B + b, mask=m)
        return carry

    lax.fori_loop(0, N // 16, step, 0)
    pltpu.async_copy(x2_hbm.at[idx_v], rows_v, sem).wait()
    pltpu.sync_copy(rows_v, xg_hbm.at[pl.ds(wid * RPW, RPW)])
    pltpu.sync_copy(idx_v, self_hbm.at[pl.ds(wid * RPW, RPW)])


@functools.cache
def _gather_call():
    return pl.kernel(
        _gather_body,
        out_type=(
            jax.ShapeDtypeStruct((BK, C), _f32),
            jax.ShapeDtypeStruct((BK,), _i32),
        ),
        mesh=_sc_mesh(),
        compiler_params=pltpu.CompilerParams(needs_layout_passes=False),
        scratch_types=[
            pltpu.VMEM((N,), _i32),
            pltpu.VMEM((RPW,), _i32),
            pltpu.VMEM((RPW, C), _f32),
            pltpu.SemaphoreType.DMA,
        ],
    )


_MT = 512


def _mm_body(xg_ref, w_ref, b_ref, z1_ref, z2_ref):
    a = xg_ref[...].astype(jnp.bfloat16)
    w = w_ref[...].astype(jnp.bfloat16)
    z = lax.dot_general(a, w, (((1,), (0,)), ((), ())),
                        preferred_element_type=_f32)
    z = jnp.maximum(z + b_ref[...], 0.0)
    z1_ref[...] = z[:, :C]
    z2_ref[...] = z[:, C:]


_mm_call = pl.pallas_call(
    _mm_body,
    grid=(BK // _MT,),
    in_specs=[
        pl.BlockSpec((_MT, C), lambda i: (i, 0)),
        pl.BlockSpec((C, 2 * C), lambda i: (0, 0)),
        pl.BlockSpec((1, 2 * C), lambda i: (0, 0)),
    ],
    out_specs=[
        pl.BlockSpec((_MT, C), lambda i: (i, 0)),
        pl.BlockSpec((_MT, C), lambda i: (i, 0)),
    ],
    out_shape=[
        jax.ShapeDtypeStruct((BK, C), _f32),
        jax.ShapeDtypeStruct((BK, C), _f32),
    ],
)


def _scatter_body(z1_hbm, z2_hbm, self_hbm, out_hbm, idx_v, idx2_v, buf, sem):
    wid = lax.axis_index("c") * 16 + lax.axis_index("s")
    base = wid * RPW
    pltpu.sync_copy(self_hbm.at[pl.ds(base, RPW)], idx_v)
    pltpu.sync_copy(z1_hbm.at[pl.ds(base, RPW)], buf)
    pltpu.async_copy(buf, out_hbm.at[idx_v], sem).wait()
    for t in range(RPW // 16):
        idx2_v[pl.ds(t * 16, 16)] = idx_v[pl.ds(t * 16, 16)] + NB
    pltpu.sync_copy(z2_hbm.at[pl.ds(base, RPW)], buf)
    pltpu.async_copy(buf, out_hbm.at[idx2_v], sem).wait()


@functools.cache
def _scatter_call():
    return pl.kernel(
        _scatter_body,
        out_type=(),
        mesh=_sc_mesh(),
        compiler_params=pltpu.CompilerParams(needs_layout_passes=False),
        scratch_types=[
            pltpu.VMEM((RPW,), _i32),
            pltpu.VMEM((RPW,), _i32),
            pltpu.VMEM((RPW, C), _f32),
            pltpu.SemaphoreType.DMA,
        ],
    )


def kernel(x, fg_score, mask, W, b):
    x2 = x.reshape(NB, C)
    base = _asm_call()(x2)
    rank3 = _rank_call(fg_score.reshape(B, 1, N), fg_score.reshape(B, N, 1),
                       mask.reshape(B, 1, N), mask.reshape(B, N, 1))
    xg, sel_flat = _gather_call()(rank3.reshape(B, N), x2)
    z1, z2 = _mm_call(xg, W, b.reshape(1, 2 * C))
    out_ref = jax.new_ref(base)
    _scatter_call()(z1, z2, sel_flat, out_ref)
    return jax.freeze(out_ref).reshape(2 * N, B, C)


# speedup vs baseline: 1.1469x; 1.1469x over previous
import functools

import jax
import jax.numpy as jnp
from jax import lax
from jax.experimental import pallas as pl
from jax.experimental.pallas import tpu as pltpu
from jax.experimental.pallas import tpu_sc as plsc

N = 2048
B = 4
C = 1024
K = 512
NB = N * B
BK = B * K
NW = 32
RPW = BK // NW
CPB = NW // B
ZPW = NB // NW

_f32 = jnp.float32
_i32 = jnp.int32


def _sc_mesh():
    return plsc.VectorSubcoreMesh(core_axis_name="c", subcore_axis_name="s")


_ACH = 32


def _asm_body(x2_hbm, out_hbm, zbuf, buf0, buf1, semz, semr, semw):
    wid = lax.axis_index("c") * 16 + lax.axis_index("s")
    base = wid * ZPW

    def zrow(r, carry):
        for l in range(C // 16):
            zbuf[r, pl.ds(l * 16, 16)] = jnp.zeros((16,), _f32)
        return carry

    lax.fori_loop(0, _ACH, zrow, 0)
    zcopies = [
        pltpu.async_copy(
            zbuf, out_hbm.at[pl.ds(NB + base + c * _ACH, _ACH)], semz)
        for c in range(ZPW // _ACH)
    ]
    bufs = [buf0, buf1]
    wrs = [None] * (ZPW // _ACH)
    for c in range(ZPW // _ACH):
        if c >= 2:
            wrs[c - 2].wait()
        pltpu.async_copy(
            x2_hbm.at[pl.ds(base + c * _ACH, _ACH)], bufs[c % 2], semr).wait()
        wrs[c] = pltpu.async_copy(
            bufs[c % 2], out_hbm.at[pl.ds(base + c * _ACH, _ACH)], semw)
    wrs[-2].wait()
    wrs[-1].wait()
    for cp in zcopies:
        cp.wait()


@functools.cache
def _asm_call():
    return pl.kernel(
        _asm_body,
        out_type=jax.ShapeDtypeStruct((2 * NB, C), _f32),
        mesh=_sc_mesh(),
        compiler_params=pltpu.CompilerParams(needs_layout_passes=False),
        scratch_types=[
            pltpu.VMEM((_ACH, C), _f32),
            pltpu.VMEM((_ACH, C), _f32),
            pltpu.VMEM((_ACH, C), _f32),
            pltpu.SemaphoreType.DMA,
            pltpu.SemaphoreType.DMA,
            pltpu.SemaphoreType.DMA,
        ],
    )


def _rank_body(s_row_ref, s_col_ref, m_row_ref, m_col_ref, rank_ref):
    neg = _f32(-jnp.inf)
    s = jnp.where(m_row_ref[0], neg, s_row_ref[0])
    sc = jnp.where(m_col_ref[0], neg, s_col_ref[0])
    jj = lax.broadcasted_iota(_i32, (1, N), 1)
    ones = jnp.ones((N, 128), _f32)
    CH = 256
    for ci in range(N // CH):
        sic = sc[ci * CH:(ci + 1) * CH, :]
        ii = lax.broadcasted_iota(_i32, (CH, 1), 0) + ci * CH
        beats = (s > sic) | ((s == sic) & (jj < ii))
        bb = beats.astype(_f32)
        cnt = lax.dot_general(bb, ones, (((1,), (0,)), ((), ())),
                              preferred_element_type=_f32)
        rank_ref[0, ci * CH:(ci + 1) * CH, :] = cnt[:, 0:1].astype(_i32)


_rank_call = pl.pallas_call(
    _rank_body,
    grid=(B,),
    in_specs=[
        pl.BlockSpec((1, 1, N), lambda i: (i, 0, 0)),
        pl.BlockSpec((1, N, 1), lambda i: (i, 0, 0)),
        pl.BlockSpec((1, 1, N), lambda i: (i, 0, 0)),
        pl.BlockSpec((1, N, 1), lambda i: (i, 0, 0)),
    ],
    out_specs=pl.BlockSpec((1, N, 1), lambda i: (i, 0, 0)),
    out_shape=jax.ShapeDtypeStruct((B, N, 1), _i32),
)


def _gather_body(rank_hbm, x2_hbm, xg_hbm, self_hbm, rank_v, idx_v, rows_v, sem):
    wid = lax.axis_index("c") * 16 + lax.axis_index("s")
    b = wid // CPB
    lo = (wid % CPB) * RPW
    pltpu.sync_copy(rank_hbm.at[b], rank_v)
    lane = lax.iota(_i32, 16)

    def step(j, carry):
        r = rank_v[pl.ds(j * 16, 16)]
        tok = lane + j * 16
        m = (r >= lo) & (r < lo + RPW)
        plsc.store_scatter(idx_v, [r - lo], tok * B + b, mask=m)
        return carry

    lax.fori_loop(0, N // 16, step, 0)
    pltpu.async_copy(x2_hbm.at[idx_v], rows_v, sem).wait()
    pltpu.sync_copy(rows_v, xg_hbm.at[pl.ds(wid * RPW, RPW)])
    pltpu.sync_copy(idx_v, self_hbm.at[pl.ds(wid * RPW, RPW)])


@functools.cache
def _gather_call():
    return pl.kernel(
        _gather_body,
        out_type=(
            jax.ShapeDtypeStruct((BK, C), _f32),
            jax.ShapeDtypeStruct((BK,), _i32),
        ),
        mesh=_sc_mesh(),
        compiler_params=pltpu.CompilerParams(needs_layout_passes=False),
        scratch_types=[
            pltpu.VMEM((N,), _i32),
            pltpu.VMEM((RPW,), _i32),
            pltpu.VMEM((RPW, C), _f32),
            pltpu.SemaphoreType.DMA,
        ],
    )


_MT = 512


def _mm_body(xg_ref, w_ref, b_ref, z1_ref, z2_ref):
    a = xg_ref[...].astype(jnp.bfloat16)
    w = w_ref[...].astype(jnp.bfloat16)
    z = lax.dot_general(a, w, (((1,), (0,)), ((), ())),
                        preferred_element_type=_f32)
    z = jnp.maximum(z + b_ref[...], 0.0)
    z1_ref[...] = z[:, :C]
    z2_ref[...] = z[:, C:]


_mm_call = pl.pallas_call(
    _mm_body,
    grid=(BK // _MT,),
    in_specs=[
        pl.BlockSpec((_MT, C), lambda i: (i, 0)),
        pl.BlockSpec((C, 2 * C), lambda i: (0, 0)),
        pl.BlockSpec((1, 2 * C), lambda i: (0, 0)),
    ],
    out_specs=[
        pl.BlockSpec((_MT, C), lambda i: (i, 0)),
        pl.BlockSpec((_MT, C), lambda i: (i, 0)),
    ],
    out_shape=[
        jax.ShapeDtypeStruct((BK, C), _f32),
        jax.ShapeDtypeStruct((BK, C), _f32),
    ],
)


def _scatter_body(z1_hbm, z2_hbm, self_hbm, out_hbm, idx_v, idx2_v, buf, sem):
    wid = lax.axis_index("c") * 16 + lax.axis_index("s")
    base = wid * RPW
    pltpu.sync_copy(self_hbm.at[pl.ds(base, RPW)], idx_v)
    pltpu.sync_copy(z1_hbm.at[pl.ds(base, RPW)], buf)
    pltpu.async_copy(buf, out_hbm.at[idx_v], sem).wait()
    for t in range(RPW // 16):
        idx2_v[pl.ds(t * 16, 16)] = idx_v[pl.ds(t * 16, 16)] + NB
    pltpu.sync_copy(z2_hbm.at[pl.ds(base, RPW)], buf)
    pltpu.async_copy(buf, out_hbm.at[idx2_v], sem).wait()


@functools.cache
def _scatter_call():
    return pl.kernel(
        _scatter_body,
        out_type=(),
        mesh=_sc_mesh(),
        compiler_params=pltpu.CompilerParams(needs_layout_passes=False),
        scratch_types=[
            pltpu.VMEM((RPW,), _i32),
            pltpu.VMEM((RPW,), _i32),
            pltpu.VMEM((RPW, C), _f32),
            pltpu.SemaphoreType.DMA,
        ],
    )


def _relay_body(lin_ref, o3_ref):
    v = lin_ref[...]
    o3_ref[...] = v.reshape(512, 4, C)


_relay_call = pl.pallas_call(
    _relay_body,
    grid=(2 * N // 512,),
    in_specs=[pl.BlockSpec((512 * 4, C), lambda i: (i, 0))],
    out_specs=pl.BlockSpec((512, 4, C), lambda i: (i, 0, 0)),
    out_shape=jax.ShapeDtypeStruct((2 * N, B, C), _f32),
)

def kernel(x, fg_score, mask, W, b):
    x2 = x.reshape(NB, C)
    base = _asm_call()(x2)
    rank3 = _rank_call(fg_score.reshape(B, 1, N), fg_score.reshape(B, N, 1),
                       mask.reshape(B, 1, N), mask.reshape(B, N, 1))
    xg, sel_flat = _gather_call()(rank3.reshape(B, N), x2)
    z1, z2 = _mm_call(xg, W, b.reshape(1, 2 * C))
    out_ref = jax.new_ref(base)
    _scatter_call()(z1, z2, sel_flat, out_ref)
    return _relay_call(jax.freeze(out_ref))


# speedup vs baseline: 1.2329x; 1.0750x over previous
import functools

import jax
import jax.numpy as jnp
from jax import lax
from jax.experimental import pallas as pl
from jax.experimental.pallas import tpu as pltpu
from jax.experimental.pallas import tpu_sc as plsc

N = 2048
B = 4
C = 1024
K = 512
NB = N * B
BK = B * K
NW = 32
RPW = BK // NW
CPB = NW // B

_f32 = jnp.float32
_i32 = jnp.int32


def _sc_mesh():
    return plsc.VectorSubcoreMesh(core_axis_name="c", subcore_axis_name="s")


def _rank_body(s_row_ref, s_col_ref, m_row_ref, m_col_ref, rank_ref, sel_ref):
    neg = _f32(-jnp.inf)
    s = jnp.where(m_row_ref[0], neg, s_row_ref[0])
    sc = jnp.where(m_col_ref[0], neg, s_col_ref[0])
    jj = lax.broadcasted_iota(_i32, (1, N), 1)
    ones = jnp.ones((N, 128), _f32)
    CH = 256
    for ci in range(N // CH):
        sic = sc[ci * CH:(ci + 1) * CH, :]
        ii = lax.broadcasted_iota(_i32, (CH, 1), 0) + ci * CH
        beats = (s > sic) | ((s == sic) & (jj < ii))
        bb = beats.astype(_f32)
        cnt = lax.dot_general(bb, ones, (((1,), (0,)), ((), ())),
                              preferred_element_type=_f32)
        cntc = cnt[:, 0:1]
        rank_ref[0, ci * CH:(ci + 1) * CH, :] = cntc.astype(_i32)
        sel_ref[ci * CH:(ci + 1) * CH, 0, 0, :] = (
            cntc < _f32(K)).astype(_f32)


_rank_call = pl.pallas_call(
    _rank_body,
    grid=(B,),
    in_specs=[
        pl.BlockSpec((1, 1, N), lambda i: (i, 0, 0)),
        pl.BlockSpec((1, N, 1), lambda i: (i, 0, 0)),
        pl.BlockSpec((1, 1, N), lambda i: (i, 0, 0)),
        pl.BlockSpec((1, N, 1), lambda i: (i, 0, 0)),
    ],
    out_specs=[
        pl.BlockSpec((1, N, 1), lambda i: (i, 0, 0)),
        pl.BlockSpec((N, 1, 1, 1), lambda i: (0, i, 0, 0)),
    ],
    out_shape=[
        jax.ShapeDtypeStruct((B, N, 1), _i32),
        jax.ShapeDtypeStruct((N, B, 1, 1), _f32),
    ],
)


def _gather_body(rank_hbm, x2_hbm, xg_hbm, self_hbm, rank_v, idx_v, rows_v, sem):
    wid = lax.axis_index("c") * 16 + lax.axis_index("s")
    b = wid // CPB
    lo = (wid % CPB) * RPW
    pltpu.sync_copy(rank_hbm.at[b], rank_v)
    lane = lax.iota(_i32, 16)

    def step(j, carry):
        r = rank_v[pl.ds(j * 16, 16)]
        tok = lane + j * 16
        m = (r >= lo) & (r < lo + RPW)
        plsc.store_scatter(idx_v, [r - lo], tok * B + b, mask=m)
        return carry

    lax.fori_loop(0, N // 16, step, 0)
    pltpu.async_copy(x2_hbm.at[idx_v], rows_v, sem).wait()
    pltpu.sync_copy(rows_v, xg_hbm.at[pl.ds(wid * RPW, RPW)])
    pltpu.sync_copy(idx_v, self_hbm.at[pl.ds(wid * RPW, RPW)])


@functools.cache
def _gather_call():
    return pl.kernel(
        _gather_body,
        out_type=(
            jax.ShapeDtypeStruct((BK, C), _f32),
            jax.ShapeDtypeStruct((BK,), _i32),
        ),
        mesh=_sc_mesh(),
        compiler_params=pltpu.CompilerParams(needs_layout_passes=False),
        scratch_types=[
            pltpu.VMEM((N,), _i32),
            pltpu.VMEM((RPW,), _i32),
            pltpu.VMEM((RPW, C), _f32),
            pltpu.SemaphoreType.DMA,
        ],
    )


_MT = 512


def _mm_body(xg_ref, w_ref, b_ref, z1_ref, z2_ref):
    a = xg_ref[...].astype(jnp.bfloat16)
    w = w_ref[...].astype(jnp.bfloat16)
    z = lax.dot_general(a, w, (((1,), (0,)), ((), ())),
                        preferred_element_type=_f32)
    z = jnp.maximum(z + b_ref[...], 0.0)
    z1_ref[...] = z[:, :C]
    z2_ref[...] = z[:, C:]


_mm_call = pl.pallas_call(
    _mm_body,
    grid=(BK // _MT,),
    in_specs=[
        pl.BlockSpec((_MT, C), lambda i: (i, 0)),
        pl.BlockSpec((C, 2 * C), lambda i: (0, 0)),
        pl.BlockSpec((1, 2 * C), lambda i: (0, 0)),
    ],
    out_specs=[
        pl.BlockSpec((_MT, C), lambda i: (i, 0)),
        pl.BlockSpec((_MT, C), lambda i: (i, 0)),
    ],
    out_shape=[
        jax.ShapeDtypeStruct((BK, C), _f32),
        jax.ShapeDtypeStruct((BK, C), _f32),
    ],
)


def _scatter_body(z1_hbm, z2_hbm, self_hbm, lin_hbm, idx_v, idx2_v, buf, sem):
    wid = lax.axis_index("c") * 16 + lax.axis_index("s")
    base = wid * RPW
    pltpu.sync_copy(self_hbm.at[pl.ds(base, RPW)], idx_v)
    pltpu.sync_copy(z1_hbm.at[pl.ds(base, RPW)], buf)
    pltpu.async_copy(buf, lin_hbm.at[idx_v], sem).wait()
    for t in range(RPW // 16):
        idx2_v[pl.ds(t * 16, 16)] = idx_v[pl.ds(t * 16, 16)] + NB
    pltpu.sync_copy(z2_hbm.at[pl.ds(base, RPW)], buf)
    pltpu.async_copy(buf, lin_hbm.at[idx2_v], sem).wait()


@functools.cache
def _scatter_call():
    return pl.kernel(
        _scatter_body,
        out_type=jax.ShapeDtypeStruct((2 * NB, C), _f32),
        mesh=_sc_mesh(),
        compiler_params=pltpu.CompilerParams(needs_layout_passes=False),
        scratch_types=[
            pltpu.VMEM((RPW,), _i32),
            pltpu.VMEM((RPW,), _i32),
            pltpu.VMEM((RPW, C), _f32),
            pltpu.SemaphoreType.DMA,
        ],
    )


_FT = 128


def _final_body(x_ref, lin1_ref, lin2_ref, sel_ref, o_ref):
    xb = x_ref[...]
    z1b = lin1_ref[...].reshape(_FT, B, C)
    z2b = lin2_ref[...].reshape(_FT, B, C)
    m = jnp.squeeze(sel_ref[...], -1) > _f32(0.5)
    o_ref[0] = jnp.where(m, z1b, xb)
    o_ref[1] = jnp.where(m, z2b, _f32(0.0))


_final_call = pl.pallas_call(
    _final_body,
    grid=(N // _FT,),
    in_specs=[
        pl.BlockSpec((_FT, B, C), lambda i: (i, 0, 0)),
        pl.BlockSpec((_FT * B, C), lambda i: (i, 0)),
        pl.BlockSpec((_FT * B, C), lambda i: (i + N // _FT, 0)),
        pl.BlockSpec((_FT, B, 1, 1), lambda i: (i, 0, 0, 0)),
    ],
    out_specs=pl.BlockSpec((2, _FT, B, C), lambda i: (0, i, 0, 0)),
    out_shape=jax.ShapeDtypeStruct((2, N, B, C), _f32),
)


def kernel(x, fg_score, mask, W, b):
    x2 = x.reshape(NB, C)
    rank3, sel4 = _rank_call(fg_score.reshape(B, 1, N), fg_score.reshape(B, N, 1),
                             mask.reshape(B, 1, N), mask.reshape(B, N, 1))
    xg, sel_flat = _gather_call()(rank3.reshape(B, N), x2)
    z1, z2 = _mm_call(xg, W, b.reshape(1, 2 * C))
    lin = _scatter_call()(z1, z2, sel_flat)
    out4 = _final_call(x, lin, lin, sel4)
    return out4.reshape(2 * N, B, C)


# speedup vs baseline: 1.2894x; 1.0458x over previous
import functools

import jax
import jax.numpy as jnp
from jax import lax
from jax.experimental import pallas as pl
from jax.experimental.pallas import tpu as pltpu
from jax.experimental.pallas import tpu_sc as plsc

N = 2048
B = 4
C = 1024
K = 512
NB = N * B
BK = B * K
NW = 32
RPW = BK // NW
CPB = NW // B

_f32 = jnp.float32
_i32 = jnp.int32


def _sc_mesh():
    return plsc.VectorSubcoreMesh(core_axis_name="c", subcore_axis_name="s")


def _rank_body(s_row_ref, s_col_ref, m_row_ref, m_col_ref, rank_ref, sel_ref):
    neg = _f32(-jnp.inf)
    s = jnp.where(m_row_ref[0], neg, s_row_ref[0])
    sc = jnp.where(m_col_ref[0], neg, s_col_ref[0])
    jj = lax.broadcasted_iota(_i32, (1, N), 1)
    ones = jnp.ones((N, 128), _f32)
    CH = 256
    for ci in range(N // CH):
        sic = sc[ci * CH:(ci + 1) * CH, :]
        ii = lax.broadcasted_iota(_i32, (CH, 1), 0) + ci * CH
        beats = (s > sic) | ((s == sic) & (jj < ii))
        bb = beats.astype(_f32)
        cnt = lax.dot_general(bb, ones, (((1,), (0,)), ((), ())),
                              preferred_element_type=_f32)
        cntc = cnt[:, 0:1]
        rank_ref[0, ci * CH:(ci + 1) * CH, :] = cntc.astype(_i32)
        sel_ref[ci * CH:(ci + 1) * CH, 0, 0, :] = (
            cntc < _f32(K)).astype(_f32)


_rank_call = pl.pallas_call(
    _rank_body,
    grid=(B,),
    in_specs=[
        pl.BlockSpec((1, 1, N), lambda i: (i, 0, 0)),
        pl.BlockSpec((1, N, 1), lambda i: (i, 0, 0)),
        pl.BlockSpec((1, 1, N), lambda i: (i, 0, 0)),
        pl.BlockSpec((1, N, 1), lambda i: (i, 0, 0)),
    ],
    out_specs=[
        pl.BlockSpec((1, N, 1), lambda i: (i, 0, 0)),
        pl.BlockSpec((N, 1, 1, 1), lambda i: (0, i, 0, 0)),
    ],
    out_shape=[
        jax.ShapeDtypeStruct((B, N, 1), _i32),
        jax.ShapeDtypeStruct((N, B, 1, 1), _f32),
    ],
)


def _gather_body(rank_hbm, x3_hbm, xg_hbm, self_hbm,
                 rank_v, idx_v, tok_v, sup_v, rows_v, sem):
    wid = lax.axis_index("c") * 16 + lax.axis_index("s")
    b = wid // CPB
    lo = (wid % CPB) * RPW
    pltpu.sync_copy(rank_hbm.at[b], rank_v)
    lane = lax.iota(_i32, 16)

    def step(j, carry):
        r = rank_v[pl.ds(j * 16, 16)]
        tok = lane + j * 16
        m = (r >= lo) & (r < lo + RPW)
        plsc.store_scatter(idx_v, [r - lo], tok * B + b, mask=m)
        plsc.store_scatter(tok_v, [r - lo], tok, mask=m)
        return carry

    lax.fori_loop(0, N // 16, step, 0)
    for g in range(RPW // 8):
        pltpu.async_copy(
            x3_hbm.at[tok_v.at[pl.ds(g * 8, 8)]], sup_v, sem).wait()

        def extract(s, carry):
            for l in range(C // 16):
                rows_v[g * 8 + s, pl.ds(l * 16, 16)] = (
                    sup_v[s, b, pl.ds(l * 16, 16)])
            return carry

        lax.fori_loop(0, 8, extract, 0)
    pltpu.sync_copy(rows_v, xg_hbm.at[pl.ds(wid * RPW, RPW)])
    pltpu.sync_copy(idx_v, self_hbm.at[pl.ds(wid * RPW, RPW)])


@functools.cache
def _gather_call():
    return pl.kernel(
        _gather_body,
        out_type=(
            jax.ShapeDtypeStruct((BK, C), _f32),
            jax.ShapeDtypeStruct((BK,), _i32),
        ),
        mesh=_sc_mesh(),
        compiler_params=pltpu.CompilerParams(needs_layout_passes=False),
        scratch_types=[
            pltpu.VMEM((N,), _i32),
            pltpu.VMEM((RPW,), _i32),
            pltpu.VMEM((RPW,), _i32),
            pltpu.VMEM((8, B, C), _f32),
            pltpu.VMEM((RPW, C), _f32),
            pltpu.SemaphoreType.DMA,
        ],
    )


_MT = 512


def _mm_body(xg_ref, w_ref, b_ref, z1_ref, z2_ref):
    a = xg_ref[...].astype(jnp.bfloat16)
    w = w_ref[...].astype(jnp.bfloat16)
    z = lax.dot_general(a, w, (((1,), (0,)), ((), ())),
                        preferred_element_type=_f32)
    z = jnp.maximum(z + b_ref[...], 0.0)
    z1_ref[...] = z[:, :C]
    z2_ref[...] = z[:, C:]


_mm_call = pl.pallas_call(
    _mm_body,
    grid=(BK // _MT,),
    in_specs=[
        pl.BlockSpec((_MT, C), lambda i: (i, 0)),
        pl.BlockSpec((C, 2 * C), lambda i: (0, 0)),
        pl.BlockSpec((1, 2 * C), lambda i: (0, 0)),
    ],
    out_specs=[
        pl.BlockSpec((_MT, C), lambda i: (i, 0)),
        pl.BlockSpec((_MT, C), lambda i: (i, 0)),
    ],
    out_shape=[
        jax.ShapeDtypeStruct((BK, C), _f32),
        jax.ShapeDtypeStruct((BK, C), _f32),
    ],
)


def _scatter_body(z1_hbm, z2_hbm, self_hbm, lin_hbm, idx_v, idx2_v, buf, sem):
    wid = lax.axis_index("c") * 16 + lax.axis_index("s")
    base = wid * RPW
    pltpu.sync_copy(self_hbm.at[pl.ds(base, RPW)], idx_v)
    pltpu.sync_copy(z1_hbm.at[pl.ds(base, RPW)], buf)
    pltpu.async_copy(buf, lin_hbm.at[idx_v], sem).wait()
    for t in range(RPW // 16):
        idx2_v[pl.ds(t * 16, 16)] = idx_v[pl.ds(t * 16, 16)] + NB
    pltpu.sync_copy(z2_hbm.at[pl.ds(base, RPW)], buf)
    pltpu.async_copy(buf, lin_hbm.at[idx2_v], sem).wait()


@functools.cache
def _scatter_call():
    return pl.kernel(
        _scatter_body,
        out_type=jax.ShapeDtypeStruct((2 * NB, C), _f32),
        mesh=_sc_mesh(),
        compiler_params=pltpu.CompilerParams(needs_layout_passes=False),
        scratch_types=[
            pltpu.VMEM((RPW,), _i32),
            pltpu.VMEM((RPW,), _i32),
            pltpu.VMEM((RPW, C), _f32),
            pltpu.SemaphoreType.DMA,
        ],
    )


_FT = 128


def _final_body(x_ref, lin1_ref, lin2_ref, sel_ref, o_ref):
    xb = x_ref[...]
    z1b = lin1_ref[...].reshape(_FT, B, C)
    z2b = lin2_ref[...].reshape(_FT, B, C)
    m = jnp.squeeze(sel_ref[...], -1) > _f32(0.5)
    o_ref[0] = jnp.where(m, z1b, xb)
    o_ref[1] = jnp.where(m, z2b, _f32(0.0))


_final_call = pl.pallas_call(
    _final_body,
    grid=(N // _FT,),
    in_specs=[
        pl.BlockSpec((_FT, B, C), lambda i: (i, 0, 0)),
        pl.BlockSpec((_FT * B, C), lambda i: (i, 0)),
        pl.BlockSpec((_FT * B, C), lambda i: (i + N // _FT, 0)),
        pl.BlockSpec((_FT, B, 1, 1), lambda i: (i, 0, 0, 0)),
    ],
    out_specs=pl.BlockSpec((2, _FT, B, C), lambda i: (0, i, 0, 0)),
    out_shape=jax.ShapeDtypeStruct((2, N, B, C), _f32),
)


def kernel(x, fg_score, mask, W, b):
    rank3, sel4 = _rank_call(fg_score.reshape(B, 1, N), fg_score.reshape(B, N, 1),
                             mask.reshape(B, 1, N), mask.reshape(B, N, 1))
    xg, sel_flat = _gather_call()(rank3.reshape(B, N), x)
    z1, z2 = _mm_call(xg, W, b.reshape(1, 2 * C))
    lin = _scatter_call()(z1, z2, sel_flat)
    out4 = _final_call(x, lin, lin, sel4)
    return out4.reshape(2 * N, B, C)


# speedup vs baseline: 1.3995x; 1.0854x over previous
import functools

import jax
import jax.numpy as jnp
from jax import lax
from jax.experimental import pallas as pl
from jax.experimental.pallas import tpu as pltpu
from jax.experimental.pallas import tpu_sc as plsc

N = 2048
B = 4
C = 1024
K = 512
NB = N * B
BK = B * K
NW = 32
RPW = BK // NW
CPB = NW // B

_f32 = jnp.float32
_i32 = jnp.int32


def _sc_mesh():
    return plsc.VectorSubcoreMesh(core_axis_name="c", subcore_axis_name="s")


def _rank_body(s_row_ref, s_col_ref, m_row_ref, m_col_ref, rank_ref, sel_ref):
    neg = _f32(-jnp.inf)
    s = jnp.where(m_row_ref[0], neg, s_row_ref[0])
    sc = jnp.where(m_col_ref[0], neg, s_col_ref[0])
    jj = lax.broadcasted_iota(_i32, (1, N), 1)
    ones = jnp.ones((N, 128), _f32)
    CH = 256
    for ci in range(N // CH):
        sic = sc[ci * CH:(ci + 1) * CH, :]
        ii = lax.broadcasted_iota(_i32, (CH, 1), 0) + ci * CH
        beats = (s > sic) | ((s == sic) & (jj < ii))
        bb = beats.astype(_f32)
        cnt = lax.dot_general(bb, ones, (((1,), (0,)), ((), ())),
                              preferred_element_type=_f32)
        cntc = cnt[:, 0:1]
        rank_ref[0, ci * CH:(ci + 1) * CH, :] = cntc.astype(_i32)
        sel_ref[ci * CH:(ci + 1) * CH, 0, 0, :] = (
            cntc < _f32(K)).astype(_f32)


_rank_call = pl.pallas_call(
    _rank_body,
    grid=(B,),
    in_specs=[
        pl.BlockSpec((1, 1, N), lambda i: (i, 0, 0)),
        pl.BlockSpec((1, N, 1), lambda i: (i, 0, 0)),
        pl.BlockSpec((1, 1, N), lambda i: (i, 0, 0)),
        pl.BlockSpec((1, N, 1), lambda i: (i, 0, 0)),
    ],
    out_specs=[
        pl.BlockSpec((1, N, 1), lambda i: (i, 0, 0)),
        pl.BlockSpec((N, 1, 1, 1), lambda i: (0, i, 0, 0)),
    ],
    out_shape=[
        jax.ShapeDtypeStruct((B, N, 1), _i32),
        jax.ShapeDtypeStruct((N, B, 1, 1), _f32),
    ],
)


def _gather_body(rank_hbm, x3_hbm, xg_hbm, self_hbm,
                 rank_v, idx_v, tok_v, sup0_v, sup1_v, rb0_v, rb1_v,
                 sem0, sem1, semw):
    wid = lax.axis_index("c") * 16 + lax.axis_index("s")
    b = wid // CPB
    lo = (wid % CPB) * RPW
    pltpu.sync_copy(rank_hbm.at[b], rank_v)
    lane = lax.iota(_i32, 16)

    def step(j, carry):
        r = rank_v[pl.ds(j * 16, 16)]
        tok = lane + j * 16
        m = (r >= lo) & (r < lo + RPW)
        plsc.store_scatter(idx_v, [r - lo], tok * B + b, mask=m)
        plsc.store_scatter(tok_v, [r - lo], tok, mask=m)
        return carry

    lax.fori_loop(0, N // 16, step, 0)
    _G = 8
    nround = RPW // _G
    sups = [sup0_v, sup1_v]
    sems = [sem0, sem1]
    rbufs = [rb0_v, rb1_v]
    cps = [None] * nround
    wcps = [None] * nround
    cps[0] = pltpu.async_copy(
        x3_hbm.at[tok_v.at[pl.ds(0, _G)]], sups[0], sems[0])
    for g in range(nround):
        cps[g].wait()
        if g + 1 < nround:
            cps[g + 1] = pltpu.async_copy(
                x3_hbm.at[tok_v.at[pl.ds((g + 1) * _G, _G)]],
                sups[(g + 1) % 2], sems[(g + 1) % 2])
        if g >= 2:
            wcps[g - 2].wait()
        sup = sups[g % 2]
        rb = rbufs[g % 2]

        def extract(s, carry, sup=sup, rb=rb):
            for l in range(C // 16):
                rb[s, pl.ds(l * 16, 16)] = sup[s, b, pl.ds(l * 16, 16)]
            return carry

        lax.fori_loop(0, _G, extract, 0)
        wcps[g] = pltpu.async_copy(
            rb, xg_hbm.at[pl.ds(wid * RPW + g * _G, _G)], semw)
    wcps[-2].wait()
    wcps[-1].wait()
    pltpu.sync_copy(idx_v, self_hbm.at[pl.ds(wid * RPW, RPW)])


@functools.cache
def _gather_call():
    return pl.kernel(
        _gather_body,
        out_type=(
            jax.ShapeDtypeStruct((BK, C), _f32),
            jax.ShapeDtypeStruct((BK,), _i32),
        ),
        mesh=_sc_mesh(),
        compiler_params=pltpu.CompilerParams(needs_layout_passes=False),
        scratch_types=[
            pltpu.VMEM((N,), _i32),
            pltpu.VMEM((RPW,), _i32),
            pltpu.VMEM((RPW,), _i32),
            pltpu.VMEM((8, B, C), _f32),
            pltpu.VMEM((8, B, C), _f32),
            pltpu.VMEM((8, C), _f32),
            pltpu.VMEM((8, C), _f32),
            pltpu.SemaphoreType.DMA,
            pltpu.SemaphoreType.DMA,
            pltpu.SemaphoreType.DMA,
        ],
    )


_MT = 512


def _mm_body(xg_ref, w_ref, b_ref, z1_ref, z2_ref):
    a = xg_ref[...].astype(jnp.bfloat16)
    w = w_ref[...].astype(jnp.bfloat16)
    z = lax.dot_general(a, w, (((1,), (0,)), ((), ())),
                        preferred_element_type=_f32)
    z = jnp.maximum(z + b_ref[...], 0.0)
    z1_ref[...] = z[:, :C]
    z2_ref[...] = z[:, C:]


_mm_call = pl.pallas_call(
    _mm_body,
    grid=(BK // _MT,),
    in_specs=[
        pl.BlockSpec((_MT, C), lambda i: (i, 0)),
        pl.BlockSpec((C, 2 * C), lambda i: (0, 0)),
        pl.BlockSpec((1, 2 * C), lambda i: (0, 0)),
    ],
    out_specs=[
        pl.BlockSpec((_MT, C), lambda i: (i, 0)),
        pl.BlockSpec((_MT, C), lambda i: (i, 0)),
    ],
    out_shape=[
        jax.ShapeDtypeStruct((BK, C), _f32),
        jax.ShapeDtypeStruct((BK, C), _f32),
    ],
)


def _scatter_body(z1_hbm, z2_hbm, self_hbm, lin_hbm, idx_v, idx2_v, buf, sem):
    wid = lax.axis_index("c") * 16 + lax.axis_index("s")
    base = wid * RPW
    pltpu.sync_copy(self_hbm.at[pl.ds(base, RPW)], idx_v)
    pltpu.sync_copy(z1_hbm.at[pl.ds(base, RPW)], buf)
    pltpu.async_copy(buf, lin_hbm.at[idx_v], sem).wait()
    for t in range(RPW // 16):
        idx2_v[pl.ds(t * 16, 16)] = idx_v[pl.ds(t * 16, 16)] + NB
    pltpu.sync_copy(z2_hbm.at[pl.ds(base, RPW)], buf)
    pltpu.async_copy(buf, lin_hbm.at[idx2_v], sem).wait()


@functools.cache
def _scatter_call():
    return pl.kernel(
        _scatter_body,
        out_type=jax.ShapeDtypeStruct((2 * NB, C), _f32),
        mesh=_sc_mesh(),
        compiler_params=pltpu.CompilerParams(needs_layout_passes=False),
        scratch_types=[
            pltpu.VMEM((RPW,), _i32),
            pltpu.VMEM((RPW,), _i32),
            pltpu.VMEM((RPW, C), _f32),
            pltpu.SemaphoreType.DMA,
        ],
    )


_FT = 128


def _final_body(x_ref, lin1_ref, lin2_ref, sel_ref, o_ref):
    xb = x_ref[...]
    z1b = lin1_ref[...].reshape(_FT, B, C)
    z2b = lin2_ref[...].reshape(_FT, B, C)
    m = jnp.squeeze(sel_ref[...], -1) > _f32(0.5)
    o_ref[0] = jnp.where(m, z1b, xb)
    o_ref[1] = jnp.where(m, z2b, _f32(0.0))


_final_call = pl.pallas_call(
    _final_body,
    grid=(N // _FT,),
    in_specs=[
        pl.BlockSpec((_FT, B, C), lambda i: (i, 0, 0)),
        pl.BlockSpec((_FT * B, C), lambda i: (i, 0)),
        pl.BlockSpec((_FT * B, C), lambda i: (i + N // _FT, 0)),
        pl.BlockSpec((_FT, B, 1, 1), lambda i: (i, 0, 0, 0)),
    ],
    out_specs=pl.BlockSpec((2, _FT, B, C), lambda i: (0, i, 0, 0)),
    out_shape=jax.ShapeDtypeStruct((2, N, B, C), _f32),
)


def kernel(x, fg_score, mask, W, b):
    rank3, sel4 = _rank_call(fg_score.reshape(B, 1, N), fg_score.reshape(B, N, 1),
                             mask.reshape(B, 1, N), mask.reshape(B, N, 1))
    xg, sel_flat = _gather_call()(rank3.reshape(B, N), x)
    z1, z2 = _mm_call(xg, W, b.reshape(1, 2 * C))
    lin = _scatter_call()(z1, z2, sel_flat)
    out4 = _final_call(x, lin, lin, sel4)
    return out4.reshape(2 * N, B, C)


# speedup vs baseline: 1.4043x; 1.0035x over previous
import functools

import jax
import jax.numpy as jnp
from jax import lax
from jax.experimental import pallas as pl
from jax.experimental.pallas import tpu as pltpu
from jax.experimental.pallas import tpu_sc as plsc

N = 2048
B = 4
C = 1024
K = 512
NB = N * B
BK = B * K
NW = 32
RPW = BK // NW
CPB = NW // B

_f32 = jnp.float32
_i32 = jnp.int32


def _sc_mesh():
    return plsc.VectorSubcoreMesh(core_axis_name="c", subcore_axis_name="s")


def _rank_body(s_row_ref, s_col_ref, m_row_ref, m_col_ref, rank_ref, sel_ref):
    neg = _f32(-jnp.inf)
    s = jnp.where(m_row_ref[0], neg, s_row_ref[0])
    sc = jnp.where(m_col_ref[0], neg, s_col_ref[0])
    jj = lax.broadcasted_iota(_i32, (1, N), 1)
    ones = jnp.ones((N, 128), _f32)
    CH = 256
    for ci in range(N // CH):
        sic = sc[ci * CH:(ci + 1) * CH, :]
        ii = lax.broadcasted_iota(_i32, (CH, 1), 0) + ci * CH
        beats = (s > sic) | ((s == sic) & (jj < ii))
        bb = beats.astype(_f32)
        cnt = lax.dot_general(bb, ones, (((1,), (0,)), ((), ())),
                              preferred_element_type=_f32)
        cntc = cnt[:, 0:1]
        rank_ref[0, ci * CH:(ci + 1) * CH, :] = cntc.astype(_i32)
        sel_ref[ci * CH:(ci + 1) * CH, 0, 0, :] = (
            cntc < _f32(K)).astype(_f32)


_rank_call = pl.pallas_call(
    _rank_body,
    grid=(B,),
    in_specs=[
        pl.BlockSpec((1, 1, N), lambda i: (i, 0, 0)),
        pl.BlockSpec((1, N, 1), lambda i: (i, 0, 0)),
        pl.BlockSpec((1, 1, N), lambda i: (i, 0, 0)),
        pl.BlockSpec((1, N, 1), lambda i: (i, 0, 0)),
    ],
    out_specs=[
        pl.BlockSpec((1, N, 1), lambda i: (i, 0, 0)),
        pl.BlockSpec((N, 1, 1, 1), lambda i: (0, i, 0, 0)),
    ],
    out_shape=[
        jax.ShapeDtypeStruct((B, N, 1), _i32),
        jax.ShapeDtypeStruct((N, B, 1, 1), _f32),
    ],
)


def _gather_body(rank_hbm, x3_hbm, xg_hbm, self_hbm,
                 rank_v, idx_v, tok_v, sup0_v, sup1_v, rb0_v, rb1_v,
                 sem0, sem1, semw):
    wid = lax.axis_index("c") * 16 + lax.axis_index("s")
    b = wid // CPB
    lo = (wid % CPB) * RPW
    pltpu.sync_copy(rank_hbm.at[b], rank_v)
    lane = lax.iota(_i32, 16)

    def step(j, carry):
        r = rank_v[pl.ds(j * 16, 16)]
        tok = lane + j * 16
        m = (r >= lo) & (r < lo + RPW)
        plsc.store_scatter(idx_v, [r - lo], tok * B + b, mask=m)
        plsc.store_scatter(tok_v, [r - lo], tok, mask=m)
        return carry

    lax.fori_loop(0, N // 16, step, 0)
    _G = 8
    nround = RPW // _G
    sups = [sup0_v, sup1_v]
    sems = [sem0, sem1]
    rbufs = [rb0_v, rb1_v]
    cps = [None] * nround
    wcps = [None] * nround
    cps[0] = pltpu.async_copy(
        x3_hbm.at[tok_v.at[pl.ds(0, _G)]], sups[0], sems[0])
    for g in range(nround):
        cps[g].wait()
        if g + 1 < nround:
            cps[g + 1] = pltpu.async_copy(
                x3_hbm.at[tok_v.at[pl.ds((g + 1) * _G, _G)]],
                sups[(g + 1) % 2], sems[(g + 1) % 2])
        if g >= 2:
            wcps[g - 2].wait()
        sup = sups[g % 2]
        rb = rbufs[g % 2]

        def extract(s, carry, sup=sup, rb=rb):
            for l in range(C // 16):
                rb[s, pl.ds(l * 16, 16)] = sup[s, b, pl.ds(l * 16, 16)]
            return carry

        lax.fori_loop(0, _G, extract, 0)
        wcps[g] = pltpu.async_copy(
            rb, xg_hbm.at[pl.ds(wid * RPW + g * _G, _G)], semw)
    wcps[-2].wait()
    wcps[-1].wait()
    pltpu.sync_copy(idx_v, self_hbm.at[pl.ds(wid * RPW, RPW)])


@functools.cache
def _gather_call():
    return pl.kernel(
        _gather_body,
        out_type=(
            jax.ShapeDtypeStruct((BK, C), _f32),
            jax.ShapeDtypeStruct((BK,), _i32),
        ),
        mesh=_sc_mesh(),
        compiler_params=pltpu.CompilerParams(needs_layout_passes=False),
        scratch_types=[
            pltpu.VMEM((N,), _i32),
            pltpu.VMEM((RPW,), _i32),
            pltpu.VMEM((RPW,), _i32),
            pltpu.VMEM((8, B, C), _f32),
            pltpu.VMEM((8, B, C), _f32),
            pltpu.VMEM((8, C), _f32),
            pltpu.VMEM((8, C), _f32),
            pltpu.SemaphoreType.DMA,
            pltpu.SemaphoreType.DMA,
            pltpu.SemaphoreType.DMA,
        ],
    )


_MT = 512


def _mm_body(xg_ref, w_ref, b_ref, z1_ref, z2_ref):
    a = xg_ref[...].astype(jnp.bfloat16)
    z = lax.dot_general(a, w_ref[...], (((1,), (0,)), ((), ())),
                        preferred_element_type=_f32)
    z = jnp.maximum(z + b_ref[...], 0.0)
    z1_ref[...] = z[:, :C]
    z2_ref[...] = z[:, C:]


_mm_call = pl.pallas_call(
    _mm_body,
    grid=(BK // _MT,),
    in_specs=[
        pl.BlockSpec((_MT, C), lambda i: (i, 0)),
        pl.BlockSpec((C, 2 * C), lambda i: (0, 0)),
        pl.BlockSpec((1, 2 * C), lambda i: (0, 0)),
    ],
    out_specs=[
        pl.BlockSpec((_MT, C), lambda i: (i, 0)),
        pl.BlockSpec((_MT, C), lambda i: (i, 0)),
    ],
    out_shape=[
        jax.ShapeDtypeStruct((BK, C), _f32),
        jax.ShapeDtypeStruct((BK, C), _f32),
    ],
)


def _scatter_body(z1_hbm, z2_hbm, self_hbm, lin_hbm, idx_v, idx2_v, buf, sem):
    wid = lax.axis_index("c") * 16 + lax.axis_index("s")
    base = wid * RPW
    pltpu.sync_copy(self_hbm.at[pl.ds(base, RPW)], idx_v)
    pltpu.sync_copy(z1_hbm.at[pl.ds(base, RPW)], buf)
    pltpu.async_copy(buf, lin_hbm.at[idx_v], sem).wait()
    for t in range(RPW // 16):
        idx2_v[pl.ds(t * 16, 16)] = idx_v[pl.ds(t * 16, 16)] + NB
    pltpu.sync_copy(z2_hbm.at[pl.ds(base, RPW)], buf)
    pltpu.async_copy(buf, lin_hbm.at[idx2_v], sem).wait()


@functools.cache
def _scatter_call():
    return pl.kernel(
        _scatter_body,
        out_type=jax.ShapeDtypeStruct((2 * NB, C), _f32),
        mesh=_sc_mesh(),
        compiler_params=pltpu.CompilerParams(needs_layout_passes=False),
        scratch_types=[
            pltpu.VMEM((RPW,), _i32),
            pltpu.VMEM((RPW,), _i32),
            pltpu.VMEM((RPW, C), _f32),
            pltpu.SemaphoreType.DMA,
        ],
    )


_FT = 128


def _final_body(x_ref, lin1_ref, lin2_ref, sel_ref, o_ref):
    xb = x_ref[...]
    z1b = lin1_ref[...].reshape(_FT, B, C)
    z2b = lin2_ref[...].reshape(_FT, B, C)
    m = jnp.squeeze(sel_ref[...], -1) > _f32(0.5)
    o_ref[0] = jnp.where(m, z1b, xb)
    o_ref[1] = jnp.where(m, z2b, _f32(0.0))


_final_call = pl.pallas_call(
    _final_body,
    grid=(N // _FT,),
    in_specs=[
        pl.BlockSpec((_FT, B, C), lambda i: (i, 0, 0)),
        pl.BlockSpec((_FT * B, C), lambda i: (i, 0)),
        pl.BlockSpec((_FT * B, C), lambda i: (i + N // _FT, 0)),
        pl.BlockSpec((_FT, B, 1, 1), lambda i: (i, 0, 0, 0)),
    ],
    out_specs=pl.BlockSpec((2, _FT, B, C), lambda i: (0, i, 0, 0)),
    out_shape=jax.ShapeDtypeStruct((2, N, B, C), _f32),
)


def kernel(x, fg_score, mask, W, b):
    rank3, sel4 = _rank_call(fg_score.reshape(B, 1, N), fg_score.reshape(B, N, 1),
                             mask.reshape(B, 1, N), mask.reshape(B, N, 1))
    xg, sel_flat = _gather_call()(rank3.reshape(B, N), x)
    z1, z2 = _mm_call(xg, W.astype(jnp.bfloat16), b.reshape(1, 2 * C))
    lin = _scatter_call()(z1, z2, sel_flat)
    out4 = _final_call(x, lin, lin, sel4)
    return out4.reshape(2 * N, B, C)
